# baseline (speedup 1.0000x reference)
import functools

import jax
import jax.numpy as jnp
from jax import lax
from jax.experimental import pallas as pl
from jax.experimental.pallas import tpu as pltpu
from jax.experimental.pallas import tpu_sc as plsc

N = 10000
E = 320000
C = 128

NC = 2
NS = 16
NW = NC * NS
EPW = E // NW
GCH = 80
NCH = EPW // GCH
RPT = 624
TAIL = N - RPT * NS
TAIL_OFF = RPT * NS

BE = 2000
BN = 2000

_F32 = jnp.float32


def _sc_mesh():
    return plsc.VectorSubcoreMesh(
        core_axis_name="c", subcore_axis_name="s", num_cores=NC, num_subcores=NS
    )


def _mm(x, w):
    return lax.dot_general(x, w, (((1,), (0,)), ((), ())),
                           preferred_element_type=_F32)


def _mmb(x, w):
    return lax.dot_general(x.astype(jnp.bfloat16), w.astype(jnp.bfloat16),
                           (((1,), (0,)), ((), ())),
                           preferred_element_type=_F32)


def _ln(w, g, b):
    mu = jnp.mean(w, axis=-1, keepdims=True)
    var = jnp.mean((w - mu) ** 2, axis=-1, keepdims=True)
    return (w - mu) * lax.rsqrt(var + 1e-5) * g + b


def _rows(bm, d):
    return pl.BlockSpec((bm, d), lambda i: (i, 0))


def _full(shape):
    return pl.BlockSpec(shape, lambda i: (0,) * len(shape))


def _node_enc_body(x_ref, w1, b1, w2, b2, w3, b3, lg, lb, wps, wpr,
                   h_ref, a_ref, p_ref):
    u = jnp.maximum(_mm(x_ref[...], w1[...]) + b1[...], 0.0)
    v = jnp.maximum(_mm(u, w2[...]) + b2[...], 0.0)
    w = _mm(v, w3[...]) + b3[...]
    h = _ln(w, lg[...], lb[...])
    h_ref[...] = h
    a_ref[...] = _mm(h, wps[...])
    p_ref[...] = _mm(h, wpr[...])


def _edge_enc_body(gs_ref, gr_ref, w8, we2, b1, w2, b2, w3, b3, lg, lb, out_ref):
    g = gs_ref[...] - gr_ref[...]
    col = lax.broadcasted_iota(jnp.int32, (1, 8), 1)
    pmask = jnp.where((col >= 4) & (col < 7), 1.0, 0.0)
    e2 = jnp.sqrt(jnp.sum(g * g * pmask, axis=1, keepdims=True))
    pre = _mm(g, w8[...]) + e2 * we2[...] + b1[...]
    u = jnp.maximum(pre, 0.0)
    v = jnp.maximum(_mmb(u, w2[...]) + b2[...], 0.0)
    w = _mmb(v, w3[...]) + b3[...]
    out_ref[...] = _ln(w, lg[...], lb[...])


def _edge_upd_body(as_ref, br_ref, e_ref, w1e, b1, w2, b2, w3, b3, lg, lb,
                   out_ref):
    e = e_ref[...]
    pre = as_ref[...] + br_ref[...] + _mmb(e, w1e[...]) + b1[...]
    u = jnp.maximum(pre, 0.0)
    v = jnp.maximum(_mmb(u, w2[...]) + b2[...], 0.0)
    w = _mmb(v, w3[...]) + b3[...]
    out_ref[...] = e + _ln(w, lg[...], lb[...])


def _node_upd_body(h_ref, p0_ref, p1_ref, w1h, w1a, b1, w2, b2, w3, b3, lg, lb,
                   wps, wpr, hn_ref, a_ref, p_ref):
    h = h_ref[...]
    agg = p0_ref[...] + p1_ref[...]
    pre = _mm(h, w1h[...]) + _mm(agg, w1a[...]) + b1[...]
    u = jnp.maximum(pre, 0.0)
    v = jnp.maximum(_mm(u, w2[...]) + b2[...], 0.0)
    w = _mm(v, w3[...]) + b3[...]
    hn = h + _ln(w, lg[...], lb[...])
    hn_ref[...] = hn
    a_ref[...] = _mm(hn, wps[...])
    p_ref[...] = _mm(hn, wpr[...])


def _node_upd_dec_body(h_ref, p0_ref, p1_ref, w1h, w1a, b1, w2, b2, w3, b3,
                       lg, lb, d1, db1, d2, db2, d3, db3, out_ref):
    h = h_ref[...]
    agg = p0_ref[...] + p1_ref[...]
    pre = _mm(h, w1h[...]) + _mm(agg, w1a[...]) + b1[...]
    u = jnp.maximum(pre, 0.0)
    v = jnp.maximum(_mm(u, w2[...]) + b2[...], 0.0)
    w = _mm(v, w3[...]) + b3[...]
    hn = h + _ln(w, lg[...], lb[...])
    du = jnp.maximum(_mm(hn, d1[...]) + db1[...], 0.0)
    dv = jnp.maximum(_mm(du, d2[...]) + db2[...], 0.0)
    out_ref[...] = _mm(dv, d3[...]) + db3[...]


def _tc_call(body, grid, in_arrays, in_specs, out_shapes, out_specs):
    return pl.pallas_call(
        body,
        grid=grid,
        in_specs=in_specs,
        out_specs=out_specs,
        out_shape=out_shapes,
    )(*in_arrays)


def _mlp_args(tp):
    arrays = list(tp)
    specs = [_full(a.shape) for a in arrays]
    return arrays, specs


def _node_encoder(x, tenc, wps, wpr):
    warr, wspec = _mlp_args(tenc + [wps, wpr])
    outs = [jax.ShapeDtypeStruct((N, C), _F32)] * 3
    return _tc_call(
        _node_enc_body, (N // BN,),
        [x] + warr, [_rows(BN, 4)] + wspec,
        outs, [_rows(BN, C)] * 3)


def _edge_encoder(gs, gr, tee):
    warr, wspec = _mlp_args(tee)
    return _tc_call(
        _edge_enc_body, (E // BE,),
        [gs, gr] + warr, [_rows(BE, 8), _rows(BE, 8)] + wspec,
        jax.ShapeDtypeStruct((E, C), _F32), _rows(BE, C))


def _edge_update(a_s, b_r, e, tup):
    warr, wspec = _mlp_args(tup)
    return _tc_call(
        _edge_upd_body, (E // BE,),
        [a_s, b_r, e] + warr, [_rows(BE, C)] * 3 + wspec,
        jax.ShapeDtypeStruct((E, C), _F32), _rows(BE, C))


def _node_update(h, p0, p1, tnu, wps, wpr):
    warr, wspec = _mlp_args(tnu + [wps, wpr])
    outs = [jax.ShapeDtypeStruct((N, C), _F32)] * 3
    return _tc_call(
        _node_upd_body, (N // BN,),
        [h, p0, p1] + warr, [_rows(BN, C)] * 3 + wspec,
        outs, [_rows(BN, C)] * 3)


def _node_update_decode(h, p0, p1, tnu, tdec):
    warr, wspec = _mlp_args(tnu + tdec)
    return _tc_call(
        _node_upd_dec_body, (N // BN,),
        [h, p0, p1] + warr, [_rows(BN, C)] * 3 + wspec,
        jax.ShapeDtypeStruct((N, 4), _F32), _rows(BN, 4))


NBUF = 5

S_GCH = 40
S_NCH = EPW // S_GCH


def _pipe_gather(tab_hbm, idx_v, bufs, out_hbm, base, gsem, wsem):
    def body(j, carry):
        gd = []
        for b in range(NBUF):
            i = j * NBUF + b
            gd.append(pltpu.async_copy(tab_hbm.at[idx_v.at[i]],
                                       bufs.at[b], gsem))
        wd = []
        for b in range(NBUF):
            i = j * NBUF + b
            gd[b].wait()
            wd.append(pltpu.async_copy(
                bufs.at[b], out_hbm.at[pl.ds(base + i * GCH, GCH)], wsem))
        for b in range(NBUF):
            wd[b].wait()
        return carry

    lax.fori_loop(0, NCH // NBUF, body, 0)


def _sc_gather_tables(table, send3, recv3):
    @functools.partial(
        pl.kernel,
        mesh=_sc_mesh(),
        out_type=[jax.ShapeDtypeStruct((E, 8), _F32)] * 2,
        scratch_types=[
            pltpu.VMEM((NCH, GCH), jnp.int32),
            pltpu.VMEM((NBUF, GCH, 8), _F32),
            pltpu.SemaphoreType.DMA,
            pltpu.SemaphoreType.DMA,
        ],
        compiler_params=pltpu.CompilerParams(use_tc_tiling_on_sc=False),
    )
    def k(tab_hbm, send_hbm, recv_hbm, gs_hbm, gr_hbm, idx_v, bufs, gsem, wsem):
        w = lax.axis_index("s") * NC + lax.axis_index("c")
        base = w * EPW
        pltpu.sync_copy(send_hbm.at[w], idx_v)
        _pipe_gather(tab_hbm, idx_v, bufs, gs_hbm, base, gsem, wsem)
        pltpu.sync_copy(recv_hbm.at[w], idx_v)
        _pipe_gather(tab_hbm, idx_v, bufs, gr_hbm, base, gsem, wsem)

    return k(table, send3, recv3)


def _sc_gather_ab(a, b, send3, recv3):
    @functools.partial(
        pl.kernel,
        mesh=_sc_mesh(),
        out_type=[jax.ShapeDtypeStruct((E, C), _F32)] * 2,
        scratch_types=[
            pltpu.VMEM((NCH, GCH), jnp.int32),
            pltpu.VMEM((NBUF, GCH, C), _F32),
            pltpu.SemaphoreType.DMA,
            pltpu.SemaphoreType.DMA,
        ],
    )
    def k(a_hbm, b_hbm, send_hbm, recv_hbm, as_hbm, br_hbm, idx_v, bufs,
          gsem, wsem):
        w = lax.axis_index("s") * NC + lax.axis_index("c")
        base = w * EPW
        pltpu.sync_copy(send_hbm.at[w], idx_v)
        _pipe_gather(a_hbm, idx_v, bufs, as_hbm, base, gsem, wsem)
        pltpu.sync_copy(recv_hbm.at[w], idx_v)
        _pipe_gather(b_hbm, idx_v, bufs, br_hbm, base, gsem, wsem)

    return k(a, b, send3, recv3)


def _sc_segment_sum(e, recv3, zeros_nc):
    @functools.partial(
        pl.kernel,
        mesh=_sc_mesh(),
        out_type=jax.ShapeDtypeStruct((NC, N, C), _F32),
        scratch_types=[
            pltpu.VMEM((NBUF, S_GCH), jnp.int32),
            pltpu.VMEM((NBUF, S_GCH, C), _F32),
            pltpu.VMEM_SHARED((N, C), _F32),
            pltpu.SemaphoreType.DMA,
            pltpu.SemaphoreType.DMA,
        ],
    )
    def k(e_hbm, recv_hbm, zero_hbm, out_hbm, idx_v, bufs, acc_sh, lsem, ssem):
        c = lax.axis_index("c")
        s = lax.axis_index("s")
        w = s * NC + c
        base = w * EPW

        pltpu.sync_copy(zero_hbm.at[pl.ds(s * RPT, RPT)],
                        acc_sh.at[pl.ds(s * RPT, RPT)])

        @pl.when(s == 0)
        def _init_tail():
            pltpu.sync_copy(zero_hbm.at[pl.ds(TAIL_OFF, TAIL)],
                            acc_sh.at[pl.ds(TAIL_OFF, TAIL)])

        plsc.subcore_barrier()

        def body(j, carry):
            pltpu.sync_copy(recv_hbm.at[w, j], idx_v)
            ld = []
            for b in range(NBUF):
                i = j * NBUF + b
                ld.append(pltpu.async_copy(
                    e_hbm.at[pl.ds(base + i * S_GCH, S_GCH)], bufs.at[b], lsem))
            sd = []
            for b in range(NBUF):
                ld[b].wait()
                sd.append(pltpu.async_copy(
                    bufs.at[b], acc_sh.at[idx_v.at[b]], ssem, add=True))
            for b in range(NBUF):
                sd[b].wait()
            return carry

        lax.fori_loop(0, S_NCH // NBUF, body, 0)

        plsc.subcore_barrier()
        pltpu.sync_copy(acc_sh.at[pl.ds(s * RPT, RPT)],
                        out_hbm.at[c, pl.ds(s * RPT, RPT)])

        @pl.when(s == 0)
        def _out_tail():
            pltpu.sync_copy(acc_sh.at[pl.ds(TAIL_OFF, TAIL)],
                            out_hbm.at[c, pl.ds(TAIL_OFF, TAIL)])

    return k(e, recv3, zeros_nc)


def _t_mlp(p):
    out = []
    for wm, bv in zip(p["W"], p["b"]):
        out.append(wm.T)
        out.append(bv.reshape(1, -1))
    flat = [out[0], out[1], out[2], out[3], out[4], out[5]]
    if p["ln"] is not None:
        flat.append(p["ln"]["g"].reshape(1, -1))
        flat.append(p["ln"]["b"].reshape(1, -1))
    return flat


def kernel(x, edge_index, pos, params):
    send3 = edge_index[0].reshape(NW, NCH, GCH)
    recv3 = edge_index[1].reshape(NW, NCH, GCH)
    recv3s = edge_index[1].reshape(NW, S_NCH // NBUF, NBUF, S_GCH)

    table8 = jnp.concatenate(
        [x, pos, jnp.zeros((N, 1), _F32)], axis=1)
    zeros_nc = jnp.zeros((N, C), _F32)

    enc = _t_mlp(params["node_encoder"])
    dec = _t_mlp(params["node_decoder"])

    eep = params["edge_encoder"]
    w1 = eep["W"][0]
    w8 = jnp.concatenate(
        [w1[:, 4:8], w1[:, 0:3], jnp.zeros((C, 1), _F32)], axis=1).T
    we2 = w1[:, 3:4].T
    tee = [w8, we2, eep["b"][0].reshape(1, -1),
           eep["W"][1].T, eep["b"][1].reshape(1, -1),
           eep["W"][2].T, eep["b"][2].reshape(1, -1),
           eep["ln"]["g"].reshape(1, -1), eep["ln"]["b"].reshape(1, -1)]

    lsplit = []
    for lp in params["layers"]:
        ew1 = lp["edge_updater"]["W"][0]
        nw1 = lp["node_updater"]["W"][0]
        eu = lp["edge_updater"]
        nu = lp["node_updater"]
        teu = [ew1[:, 2 * C:].T, eu["b"][0].reshape(1, -1),
               eu["W"][1].T, eu["b"][1].reshape(1, -1),
               eu["W"][2].T, eu["b"][2].reshape(1, -1),
               eu["ln"]["g"].reshape(1, -1), eu["ln"]["b"].reshape(1, -1)]
        tnu = [nw1[:, :C].T, nw1[:, C:].T, nu["b"][0].reshape(1, -1),
               nu["W"][1].T, nu["b"][1].reshape(1, -1),
               nu["W"][2].T, nu["b"][2].reshape(1, -1),
               nu["ln"]["g"].reshape(1, -1), nu["ln"]["b"].reshape(1, -1)]
        lsplit.append({
            "wps": ew1[:, :C].T,
            "wpr": ew1[:, C:2 * C].T,
            "teu": teu,
            "tnu": tnu,
        })

    gs, gr = _sc_gather_tables(table8, send3, recv3)
    e = _edge_encoder(gs, gr, tee)
    h, a, b = _node_encoder(x, enc, lsplit[0]["wps"], lsplit[0]["wpr"])

    a_s, b_r = _sc_gather_ab(a, b, send3, recv3)
    e = _edge_update(a_s, b_r, e, lsplit[0]["teu"])
    parts = _sc_segment_sum(e, recv3s, zeros_nc)
    h, a, b = _node_update(h, parts[0], parts[1], lsplit[0]["tnu"],
                           lsplit[1]["wps"], lsplit[1]["wpr"])

    a_s, b_r = _sc_gather_ab(a, b, send3, recv3)
    e = _edge_update(a_s, b_r, e, lsplit[1]["teu"])
    parts = _sc_segment_sum(e, recv3s, zeros_nc)
    return _node_update_decode(h, parts[0], parts[1], lsplit[1]["tnu"], dec)


# speedup vs baseline: 1.1128x; 1.1128x over previous
import functools

import jax
import jax.numpy as jnp
from jax import lax
from jax.experimental import pallas as pl
from jax.experimental.pallas import tpu as pltpu
from jax.experimental.pallas import tpu_sc as plsc

N = 10000
E = 320000
C = 128
EH = E // 2

NC = 2
NS = 16
NW = NC * NS
EPW = EH // NW
GCH = 40
NCH = EPW // GCH
NBUF = 5
RPT = 624
TAIL = N - RPT * NS
TAIL_OFF = RPT * NS

BE = 2000
BN = 2000

_F32 = jnp.float32


def _sc_mesh():
    return plsc.VectorSubcoreMesh(
        core_axis_name="c", subcore_axis_name="s", num_cores=NC, num_subcores=NS
    )


def _mm(x, w):
    return lax.dot_general(x, w, (((1,), (0,)), ((), ())),
                           preferred_element_type=_F32)


def _ln(w, g, b):
    mu = jnp.mean(w, axis=-1, keepdims=True)
    var = jnp.mean((w - mu) ** 2, axis=-1, keepdims=True)
    return (w - mu) * lax.rsqrt(var + 1e-5) * g + b


def _rows(bm, d):
    return pl.BlockSpec((bm, d), lambda i: (i, 0))


def _full(shape):
    return pl.BlockSpec(shape, lambda i: (0,) * len(shape))


def _node_enc_body(x_ref, w1, b1, w2, b2, w3, b3, lg, lb, wps, wpr,
                   h_ref, a_ref, p_ref):
    u = jnp.maximum(_mm(x_ref[...], w1[...]) + b1[...], 0.0)
    v = jnp.maximum(_mm(u, w2[...]) + b2[...], 0.0)
    w = _mm(v, w3[...]) + b3[...]
    h = _ln(w, lg[...], lb[...])
    h_ref[...] = h
    a_ref[...] = _mm(h, wps[...])
    p_ref[...] = _mm(h, wpr[...])


def _edge_enc_body(gs_ref, gr_ref, w8, we2, b1, w2, b2, w3, b3, lg, lb, out_ref):
    g = gs_ref[...] - gr_ref[...]
    col = lax.broadcasted_iota(jnp.int32, (1, 8), 1)
    pmask = jnp.where((col >= 4) & (col < 7), 1.0, 0.0)
    e2 = jnp.sqrt(jnp.sum(g * g * pmask, axis=1, keepdims=True))
    pre = _mm(g, w8[...]) + e2 * we2[...] + b1[...]
    u = jnp.maximum(pre, 0.0)
    v = jnp.maximum(_mm(u, w2[...]) + b2[...], 0.0)
    w = _mm(v, w3[...]) + b3[...]
    out_ref[...] = _ln(w, lg[...], lb[...])


def _edge_upd_body(as_ref, br_ref, e_ref, w1e, b1, w2, b2, w3, b3, lg, lb,
                   out_ref):
    e = e_ref[...]
    pre = as_ref[...] + br_ref[...] + _mm(e, w1e[...]) + b1[...]
    u = jnp.maximum(pre, 0.0)
    v = jnp.maximum(_mm(u, w2[...]) + b2[...], 0.0)
    w = _mm(v, w3[...]) + b3[...]
    out_ref[...] = e + _ln(w, lg[...], lb[...])


def _node_upd_body(h_ref, p0_ref, p1_ref, p2_ref, p3_ref,
                   w1h, w1a, b1, w2, b2, w3, b3, lg, lb,
                   wps, wpr, hn_ref, a_ref, p_ref):
    h = h_ref[...]
    agg = (p0_ref[...] + p1_ref[...]) + (p2_ref[...] + p3_ref[...])
    pre = _mm(h, w1h[...]) + _mm(agg, w1a[...]) + b1[...]
    u = jnp.maximum(pre, 0.0)
    v = jnp.maximum(_mm(u, w2[...]) + b2[...], 0.0)
    w = _mm(v, w3[...]) + b3[...]
    hn = h + _ln(w, lg[...], lb[...])
    hn_ref[...] = hn
    a_ref[...] = _mm(hn, wps[...])
    p_ref[...] = _mm(hn, wpr[...])


def _node_upd_dec_body(h_ref, p0_ref, p1_ref, p2_ref, p3_ref,
                       w1h, w1a, b1, w2, b2, w3, b3,
                       lg, lb, d1, db1, d2, db2, d3, db3, out_ref):
    h = h_ref[...]
    agg = (p0_ref[...] + p1_ref[...]) + (p2_ref[...] + p3_ref[...])
    pre = _mm(h, w1h[...]) + _mm(agg, w1a[...]) + b1[...]
    u = jnp.maximum(pre, 0.0)
    v = jnp.maximum(_mm(u, w2[...]) + b2[...], 0.0)
    w = _mm(v, w3[...]) + b3[...]
    hn = h + _ln(w, lg[...], lb[...])
    du = jnp.maximum(_mm(hn, d1[...]) + db1[...], 0.0)
    dv = jnp.maximum(_mm(du, d2[...]) + db2[...], 0.0)
    out_ref[...] = _mm(dv, d3[...]) + db3[...]


def _tc_call(body, grid, in_arrays, in_specs, out_shapes, out_specs):
    return pl.pallas_call(
        body,
        grid=grid,
        in_specs=in_specs,
        out_specs=out_specs,
        out_shape=out_shapes,
    )(*in_arrays)


def _mlp_args(tp):
    arrays = list(tp)
    specs = [_full(a.shape) for a in arrays]
    return arrays, specs


def _node_encoder(x, tenc, wps, wpr):
    warr, wspec = _mlp_args(tenc + [wps, wpr])
    outs = [jax.ShapeDtypeStruct((N, C), _F32)] * 3
    return _tc_call(
        _node_enc_body, (N // BN,),
        [x] + warr, [_rows(BN, 4)] + wspec,
        outs, [_rows(BN, C)] * 3)


def _edge_encoder(gs, gr, tee):
    warr, wspec = _mlp_args(tee)
    return _tc_call(
        _edge_enc_body, (EH // BE,),
        [gs, gr] + warr, [_rows(BE, 8), _rows(BE, 8)] + wspec,
        jax.ShapeDtypeStruct((EH, C), _F32), _rows(BE, C))


def _edge_update(a_s, b_r, e, tup):
    warr, wspec = _mlp_args(tup)
    return _tc_call(
        _edge_upd_body, (EH // BE,),
        [a_s, b_r, e] + warr, [_rows(BE, C)] * 3 + wspec,
        jax.ShapeDtypeStruct((EH, C), _F32), _rows(BE, C))


def _node_update(h, parts, tnu, wps, wpr):
    warr, wspec = _mlp_args(tnu + [wps, wpr])
    outs = [jax.ShapeDtypeStruct((N, C), _F32)] * 3
    return _tc_call(
        _node_upd_body, (N // BN,),
        [h] + parts + warr, [_rows(BN, C)] * 5 + wspec,
        outs, [_rows(BN, C)] * 3)


def _node_update_decode(h, parts, tnu, tdec):
    warr, wspec = _mlp_args(tnu + tdec)
    return _tc_call(
        _node_upd_dec_body, (N // BN,),
        [h] + parts + warr, [_rows(BN, C)] * 5 + wspec,
        jax.ShapeDtypeStruct((N, 4), _F32), _rows(BN, 4))


def _pipe_gather(tab_hbm, idx_v, bufs, out_hbm, base, gsem, wsem):
    def body(j, carry):
        gd = []
        for b in range(NBUF):
            i = j * NBUF + b
            gd.append(pltpu.async_copy(
                tab_hbm.at[idx_v.at[pl.ds(i * GCH, GCH)]], bufs.at[b], gsem))
        wd = []
        for b in range(NBUF):
            i = j * NBUF + b
            gd[b].wait()
            wd.append(pltpu.async_copy(
                bufs.at[b], out_hbm.at[pl.ds(base + i * GCH, GCH)], wsem))
        for b in range(NBUF):
            wd[b].wait()
        return carry

    lax.fori_loop(0, NCH // NBUF, body, 0)


def _sc_gather_tables(table, send, recv, eoff):
    @functools.partial(
        pl.kernel,
        mesh=_sc_mesh(),
        out_type=[jax.ShapeDtypeStruct((EH, 8), _F32)] * 2,
        scratch_types=[
            pltpu.VMEM((EPW,), jnp.int32),
            pltpu.VMEM((NBUF, GCH, 8), _F32),
            pltpu.SemaphoreType.DMA,
            pltpu.SemaphoreType.DMA,
        ],
        compiler_params=pltpu.CompilerParams(use_tc_tiling_on_sc=False),
    )
    def k(tab_hbm, send_hbm, recv_hbm, gs_hbm, gr_hbm, idx_v, bufs, gsem, wsem):
        w = lax.axis_index("s") * NC + lax.axis_index("c")
        base = w * EPW
        pltpu.sync_copy(send_hbm.at[pl.ds(eoff + base, EPW)], idx_v)
        _pipe_gather(tab_hbm, idx_v, bufs, gs_hbm, base, gsem, wsem)
        pltpu.sync_copy(recv_hbm.at[pl.ds(eoff + base, EPW)], idx_v)
        _pipe_gather(tab_hbm, idx_v, bufs, gr_hbm, base, gsem, wsem)

    return k(table, send, recv)


def _sc_gather_ab(a, b, send, recv, eoff):
    @functools.partial(
        pl.kernel,
        mesh=_sc_mesh(),
        out_type=[jax.ShapeDtypeStruct((EH, C), _F32)] * 2,
        scratch_types=[
            pltpu.VMEM((EPW,), jnp.int32),
            pltpu.VMEM((NBUF, GCH, C), _F32),
            pltpu.SemaphoreType.DMA,
            pltpu.SemaphoreType.DMA,
        ],
    )
    def k(a_hbm, b_hbm, send_hbm, recv_hbm, as_hbm, br_hbm, idx_v, bufs,
          gsem, wsem):
        w = lax.axis_index("s") * NC + lax.axis_index("c")
        base = w * EPW
        pltpu.sync_copy(send_hbm.at[pl.ds(eoff + base, EPW)], idx_v)
        _pipe_gather(a_hbm, idx_v, bufs, as_hbm, base, gsem, wsem)
        pltpu.sync_copy(recv_hbm.at[pl.ds(eoff + base, EPW)], idx_v)
        _pipe_gather(b_hbm, idx_v, bufs, br_hbm, base, gsem, wsem)

    return k(a, b, send, recv)


def _sc_segment_sum(e, recv, zeros_nc, eoff):
    @functools.partial(
        pl.kernel,
        mesh=_sc_mesh(),
        out_type=jax.ShapeDtypeStruct((NC, N, C), _F32),
        scratch_types=[
            pltpu.VMEM((NBUF, GCH), jnp.int32),
            pltpu.VMEM((NBUF, GCH, C), _F32),
            pltpu.VMEM_SHARED((N, C), _F32),
            pltpu.SemaphoreType.DMA,
            pltpu.SemaphoreType.DMA,
            pltpu.SemaphoreType.DMA,
        ],
    )
    def k(e_hbm, recv_hbm, zero_hbm, out_hbm, idx_v, bufs, acc_sh,
          lsem, ssem, isem):
        c = lax.axis_index("c")
        s = lax.axis_index("s")
        w = s * NC + c
        base = w * EPW

        pltpu.sync_copy(zero_hbm.at[pl.ds(s * RPT, RPT)],
                        acc_sh.at[pl.ds(s * RPT, RPT)])

        @pl.when(s == 0)
        def _init_tail():
            pltpu.sync_copy(zero_hbm.at[pl.ds(TAIL_OFF, TAIL)],
                            acc_sh.at[pl.ds(TAIL_OFF, TAIL)])

        plsc.subcore_barrier()

        def body(j, carry):
            idxd = []
            ld = []
            for b in range(NBUF):
                i = j * NBUF + b
                idxd.append(pltpu.async_copy(
                    recv_hbm.at[pl.ds(eoff + base + i * GCH, GCH)],
                    idx_v.at[b], isem))
                ld.append(pltpu.async_copy(
                    e_hbm.at[pl.ds(base + i * GCH, GCH)], bufs.at[b], lsem))
            sd = []
            for b in range(NBUF):
                idxd[b].wait()
                ld[b].wait()
                sd.append(pltpu.async_copy(
                    bufs.at[b], acc_sh.at[idx_v.at[b]], ssem, add=True))
            for b in range(NBUF):
                sd[b].wait()
            return carry

        lax.fori_loop(0, NCH // NBUF, body, 0)

        plsc.subcore_barrier()
        pltpu.sync_copy(acc_sh.at[pl.ds(s * RPT, RPT)],
                        out_hbm.at[c, pl.ds(s * RPT, RPT)])

        @pl.when(s == 0)
        def _out_tail():
            pltpu.sync_copy(acc_sh.at[pl.ds(TAIL_OFF, TAIL)],
                            out_hbm.at[c, pl.ds(TAIL_OFF, TAIL)])

    return k(e, recv, zeros_nc)


def _t_mlp(p):
    out = []
    for wm, bv in zip(p["W"], p["b"]):
        out.append(wm.T)
        out.append(bv.reshape(1, -1))
    flat = [out[0], out[1], out[2], out[3], out[4], out[5]]
    if p["ln"] is not None:
        flat.append(p["ln"]["g"].reshape(1, -1))
        flat.append(p["ln"]["b"].reshape(1, -1))
    return flat


def kernel(x, edge_index, pos, params):
    send_f = edge_index[0]
    recv_f = edge_index[1]

    table8 = jnp.concatenate(
        [x, pos, jnp.zeros((N, 1), _F32)], axis=1)
    zeros_nc = jnp.zeros((N, C), _F32)

    enc = _t_mlp(params["node_encoder"])
    dec = _t_mlp(params["node_decoder"])

    eep = params["edge_encoder"]
    w1 = eep["W"][0]
    w8 = jnp.concatenate(
        [w1[:, 4:8], w1[:, 0:3], jnp.zeros((C, 1), _F32)], axis=1).T
    we2 = w1[:, 3:4].T
    tee = [w8, we2, eep["b"][0].reshape(1, -1),
           eep["W"][1].T, eep["b"][1].reshape(1, -1),
           eep["W"][2].T, eep["b"][2].reshape(1, -1),
           eep["ln"]["g"].reshape(1, -1), eep["ln"]["b"].reshape(1, -1)]

    lsplit = []
    for lp in params["layers"]:
        ew1 = lp["edge_updater"]["W"][0]
        nw1 = lp["node_updater"]["W"][0]
        eu = lp["edge_updater"]
        nu = lp["node_updater"]
        teu = [ew1[:, 2 * C:].T, eu["b"][0].reshape(1, -1),
               eu["W"][1].T, eu["b"][1].reshape(1, -1),
               eu["W"][2].T, eu["b"][2].reshape(1, -1),
               eu["ln"]["g"].reshape(1, -1), eu["ln"]["b"].reshape(1, -1)]
        tnu = [nw1[:, :C].T, nw1[:, C:].T, nu["b"][0].reshape(1, -1),
               nu["W"][1].T, nu["b"][1].reshape(1, -1),
               nu["W"][2].T, nu["b"][2].reshape(1, -1),
               nu["ln"]["g"].reshape(1, -1), nu["ln"]["b"].reshape(1, -1)]
        lsplit.append({
            "wps": ew1[:, :C].T,
            "wpr": ew1[:, C:2 * C].T,
            "teu": teu,
            "tnu": tnu,
        })

    h, a, b = _node_encoder(x, enc, lsplit[0]["wps"], lsplit[0]["wpr"])

    e_h = [None, None]
    for hf in range(2):
        gs, gr = _sc_gather_tables(table8, send_f, recv_f, hf * EH)
        e_h[hf] = _edge_encoder(gs, gr, tee)

    for li in range(2):
        parts = []
        ab = [ _sc_gather_ab(a, b, send_f, recv_f, hf * EH) for hf in range(2) ]
        for hf in range(2):
            a_s, b_r = ab[hf]
            e_h[hf] = _edge_update(a_s, b_r, e_h[hf], lsplit[li]["teu"])
            ps = _sc_segment_sum(e_h[hf], recv_f, zeros_nc, hf * EH)
            parts.extend([ps[0], ps[1]])
        if li == 0:
            h, a, b = _node_update(h, parts, lsplit[0]["tnu"],
                                   lsplit[1]["wps"], lsplit[1]["wpr"])
    return _node_update_decode(h, parts, lsplit[1]["tnu"], dec)
